# Initial kernel scaffold; baseline (speedup 1.0000x reference)
import jax
import jax.numpy as jnp
from jax.experimental import pallas as pl


def kernel(inputs, V):
    raise NotImplementedError("write your pallas kernel here")


# speedup vs baseline: 1.1025x; 1.1025x over previous
import functools

import jax
import jax.numpy as jnp
from jax import lax
from jax.experimental import pallas as pl
from jax.experimental.pallas import tpu as pltpu
from jax.experimental.pallas import tpu_sc as plsc

_VOCAB = 1000000
_EMB = 32
_BATCH = 16384
_HIST = 50
_B = _BATCH * _HIST
_NC = 2
_NS = 16
_NW = _NC * _NS
_BPW = _B // _NW
_CHUNK = 1600
_NCHUNK = _BPW // _CHUNK


@functools.partial(
    pl.kernel,
    mesh=plsc.VectorSubcoreMesh(core_axis_name="c", subcore_axis_name="s"),
    out_type=jax.ShapeDtypeStruct((_B, _EMB), jnp.float32),
    scratch_types=[
        pltpu.VMEM((_CHUNK,), jnp.int32),
        pltpu.VMEM((_CHUNK, _EMB), jnp.float32),
        pltpu.SemaphoreType.DMA,
    ],
    compiler_params=pltpu.CompilerParams(use_tc_tiling_on_sc=False),
)
def _sc_gather(idx_hbm, table_hbm, out_hbm, idx_v, rows_v, sem):
    wid = lax.axis_index("s") * _NC + lax.axis_index("c")
    base = wid * _BPW

    def body(i, carry):
        off = base + i * _CHUNK
        pltpu.sync_copy(idx_hbm.at[pl.ds(off, _CHUNK)], idx_v)
        pltpu.async_copy(table_hbm.at[idx_v], rows_v, sem).wait()
        pltpu.sync_copy(rows_v, out_hbm.at[pl.ds(off, _CHUNK)])
        return carry

    lax.fori_loop(0, _NCHUNK, body, 0)


def kernel(inputs, V):
    flat_idx = inputs.reshape(_B)
    out = _sc_gather(flat_idx, V)
    return out.reshape(_BATCH, _HIST, _EMB)


# speedup vs baseline: 1.1132x; 1.0097x over previous
import functools

import jax
import jax.numpy as jnp
from jax import lax
from jax.experimental import pallas as pl
from jax.experimental.pallas import tpu as pltpu
from jax.experimental.pallas import tpu_sc as plsc

_VOCAB = 1000000
_EMB = 32
_BATCH = 16384
_HIST = 50
_B = _BATCH * _HIST
_NC = 2
_NS = 16
_NW = _NC * _NS
_BPW = _B // _NW
_CHUNK = 800
_NCHUNK = _BPW // _CHUNK
_NB = 4
_NSTEP = _NCHUNK // _NB


@functools.partial(
    pl.kernel,
    mesh=plsc.VectorSubcoreMesh(core_axis_name="c", subcore_axis_name="s"),
    out_type=jax.ShapeDtypeStruct((_B, _EMB), jnp.float32),
    scratch_types=(
        [pltpu.VMEM((_CHUNK,), jnp.int32) for _ in range(_NB)]
        + [pltpu.VMEM((_CHUNK, _EMB), jnp.float32) for _ in range(_NB)]
        + [pltpu.SemaphoreType.DMA for _ in range(2 * _NB)]
    ),
    compiler_params=pltpu.CompilerParams(use_tc_tiling_on_sc=False),
)
def _sc_gather(idx_hbm, table_hbm, out_hbm, *scratch):
    idx_v = scratch[:_NB]
    rows_v = scratch[_NB:2 * _NB]
    g_sem = scratch[2 * _NB:3 * _NB]
    st_sem = scratch[3 * _NB:4 * _NB]
    wid = lax.axis_index("s") * _NC + lax.axis_index("c")
    base = wid * _BPW

    def load_idx(i, b):
        pltpu.sync_copy(idx_hbm.at[pl.ds(base + i * _CHUNK, _CHUNK)],
                        idx_v[b])

    def start_gather(b):
        pltpu.async_copy(table_hbm.at[idx_v[b]], rows_v[b], g_sem[b])

    def wait_gather(b):
        pltpu.make_async_copy(out_hbm.at[pl.ds(0, _CHUNK)], rows_v[b],
                              g_sem[b]).wait()

    def start_store(i, b):
        pltpu.async_copy(rows_v[b], out_hbm.at[pl.ds(base + i * _CHUNK, _CHUNK)],
                         st_sem[b])

    def wait_store(b):
        pltpu.make_async_copy(rows_v[b], out_hbm.at[pl.ds(0, _CHUNK)],
                              st_sem[b]).wait()

    for b in range(_NB):
        load_idx(b, b)
        start_gather(b)

    def body(j, carry):
        for b in range(_NB):
            i = j * _NB + b
            wait_gather(b)
            start_store(i, b)
            load_idx(i + _NB, b)
            wait_store(b)
            start_gather(b)
        return carry

    lax.fori_loop(0, _NSTEP - 1, body, 0)

    for b in range(_NB):
        i = (_NSTEP - 1) * _NB + b
        wait_gather(b)
        start_store(i, b)
    for b in range(_NB):
        wait_store(b)


def kernel(inputs, V):
    flat_idx = inputs.reshape(_B)
    out = _sc_gather(flat_idx, V)
    return out.reshape(_BATCH, _HIST, _EMB)
